# baseline (speedup 1.0000x reference)
import functools

import jax
import jax.numpy as jnp
from jax import lax
from jax.experimental import pallas as pl
from jax.experimental.pallas import tpu as pltpu
from jax.experimental.pallas import tpu_sc as plsc

N_NODES = 50000
D = 128
N_SEG = 8192
L = 131072
AUG = 144

_BLK = 512
_GRID = (N_NODES + _BLK - 1) // _BLK


def _tc_body(hs_ref, hf_ref, Whs_ref, bhs_ref, Whf_ref, bhf_ref,
             wphs_ref, wphf_ref, tfhs_ref, tfhf_ref, zhs_ref, zhf_ref):
    def one(x_ref, W_ref, b_ref, wp_ref, tf_ref, z_ref):
        x = x_ref[...]
        t = x + jnp.maximum(x @ W_ref[...] + b_ref[...], 0.0)
        tf_ref[...] = t
        e = jnp.exp(t @ wp_ref[...])
        z_ref[:, 0:D] = t * e
        z_ref[:, D:AUG] = jnp.broadcast_to(e, (t.shape[0], AUG - D))

    one(hs_ref, Whs_ref, bhs_ref, wphs_ref, tfhs_ref, zhs_ref)
    one(hf_ref, Whf_ref, bhf_ref, wphf_ref, tfhf_ref, zhf_ref)


def _tc_stage(hs, hf, W_hs, b_hs, W_hf, b_hf, wp_hs, wp_hf):
    row_spec = pl.BlockSpec((_BLK, D), lambda i: (i, 0))
    full = lambda shape: pl.BlockSpec(shape, lambda i: (0, 0))
    return pl.pallas_call(
        _tc_body,
        grid=(_GRID,),
        in_specs=[row_spec, row_spec,
                  full((D, D)), full((1, D)), full((D, D)), full((1, D)),
                  full((D, 1)), full((D, 1))],
        out_specs=[row_spec, row_spec,
                   pl.BlockSpec((_BLK, AUG), lambda i: (i, 0)),
                   pl.BlockSpec((_BLK, AUG), lambda i: (i, 0))],
        out_shape=[jax.ShapeDtypeStruct((N_NODES, D), jnp.float32),
                   jax.ShapeDtypeStruct((N_NODES, D), jnp.float32),
                   jax.ShapeDtypeStruct((N_NODES, AUG), jnp.float32),
                   jax.ShapeDtypeStruct((N_NODES, AUG), jnp.float32)],
    )(hs, hf, W_hs, b_hs, W_hf, b_hf, wp_hs, wp_hf)


_NS = 16
_CHUNK = 128
_EPT = L // _NS
_NCHUNK = _EPT // _CHUNK
_SPT = N_SEG // _NS
_DIVQ = 128
_NDIVQ = _SPT // _DIVQ


_NBUF = 2


def _sc_body(tbl_hbm, idx_hbm, seg_hbm, hop_hbm,
             idx_bufs, seg_bufs, row_bufs, outq_v, acc_sh, g_sems, s_sems):
    sid = lax.axis_index("s")
    z16 = jnp.zeros((16,), jnp.float32)

    rows0 = row_bufs[0]
    def zrow(i, _):
        r = i // (AUG // 16)
        k = i % (AUG // 16)
        rows0[r, pl.ds(k * 16, 16)] = z16
        return 0
    lax.fori_loop(0, _CHUNK * (AUG // 16), zrow, 0)
    def zstripe(q, _):
        pltpu.sync_copy(rows0, acc_sh.at[pl.ds((sid * (_SPT // _CHUNK) + q) * _CHUNK, _CHUNK)])
        return 0
    lax.fori_loop(0, _SPT // _CHUNK, zstripe, 0)
    plsc.subcore_barrier()

    def process(tbl_hbm, hop_hbm):
        def load_and_gather(j, b):
            base = sid * _EPT + j * _CHUNK
            pltpu.sync_copy(idx_hbm.at[pl.ds(base, _CHUNK)], idx_bufs[b])
            pltpu.sync_copy(seg_hbm.at[pl.ds(base, _CHUNK)], seg_bufs[b])
            return pltpu.async_copy(tbl_hbm.at[idx_bufs[b]], row_bufs[b],
                                    g_sems[b])

        for b in range(_NBUF):
            load_and_gather(b, b)

        def step(g, _):
            for b in range(_NBUF):
                j = g * _NBUF + b
                pltpu.make_async_copy(tbl_hbm.at[idx_bufs[b]], row_bufs[b],
                                      g_sems[b]).wait()
                pltpu.sync_copy(row_bufs[b], acc_sh.at[seg_bufs[b]], add=True)
                @pl.when(j + _NBUF < _NCHUNK)
                def _():
                    load_and_gather(j + _NBUF, b)
            return 0
        lax.fori_loop(0, _NCHUNK // _NBUF, step, 0)
        plsc.subcore_barrier()

        def divq(q, _):
            seg0 = sid * _SPT + q * _DIVQ
            pltpu.sync_copy(acc_sh.at[pl.ds(seg0, _DIVQ)], rows0)
            def seg_body(r, _):
                den_v = rows0[r, pl.ds(D, 16)] + 1e-30
                def col(k, _):
                    outq_v[r, pl.ds(k * 16, 16)] = rows0[r, pl.ds(k * 16, 16)] / den_v
                    return 0
                lax.fori_loop(0, D // 16, col, 0)
                return 0
            lax.fori_loop(0, _DIVQ, seg_body, 0)
            pltpu.sync_copy(outq_v, hop_hbm.at[pl.ds(seg0, _DIVQ)])
            return 0
        lax.fori_loop(0, _NDIVQ, divq, 0)

    process(tbl_hbm, hop_hbm)


@functools.cache
def _sc_stage():
    return pl.kernel(
        _sc_body,
        out_type=jax.ShapeDtypeStruct((N_SEG, D), jnp.float32),
        mesh=plsc.VectorSubcoreMesh(core_axis_name="c", subcore_axis_name="s",
                                    num_cores=1),
        scratch_types=[
            [pltpu.VMEM((_CHUNK,), jnp.int32) for _ in range(_NBUF)],
            [pltpu.VMEM((_CHUNK,), jnp.int32) for _ in range(_NBUF)],
            [pltpu.VMEM((_CHUNK, AUG), jnp.float32) for _ in range(_NBUF)],
            pltpu.VMEM((_DIVQ, D), jnp.float32),
            pltpu.VMEM_SHARED((N_SEG, AUG), jnp.float32),
            [pltpu.SemaphoreType.DMA for _ in range(_NBUF)],
            [pltpu.SemaphoreType.DMA for _ in range(_NBUF)],
        ],
        compiler_params=pltpu.CompilerParams(use_tc_tiling_on_sc=False),
    )


def kernel(hs, hf, flat_idx, segment_ids, W_hs, b_hs, W_hf, b_hf,
           w_pool_hs, w_pool_hf):
    idx = flat_idx.astype(jnp.int32)
    seg = segment_ids.astype(jnp.int32)
    tf_hs, tf_hf, z_hs, z_hf = _tc_stage(
        hs, hf, W_hs, b_hs.reshape(1, D), W_hf, b_hf.reshape(1, D),
        w_pool_hs.reshape(D, 1), w_pool_hf.reshape(D, 1))
    hop_hs = _sc_stage()(z_hs, idx, seg)
    hop_hf = _sc_stage()(z_hf, idx, seg)
    return tf_hs, tf_hf, hop_hs, hop_hf


# speedup vs baseline: 1.3655x; 1.3655x over previous
import functools

import jax
import jax.numpy as jnp
from jax import lax
from jax.experimental import pallas as pl
from jax.experimental.pallas import tpu as pltpu
from jax.experimental.pallas import tpu_sc as plsc

N_NODES = 50000
D = 128
N_SEG = 8192
L = 131072
AUG = 144

_BLK = 512
_GRID = (N_NODES + _BLK - 1) // _BLK


def _tc_body(hs_ref, hf_ref, Whs_ref, bhs_ref, Whf_ref, bhf_ref,
             wphs_ref, wphf_ref, tfhs_ref, tfhf_ref, zhs_ref, zhf_ref):
    def one(x_ref, W_ref, b_ref, wp_ref, tf_ref, z_ref):
        x = x_ref[...]
        t = x + jnp.maximum(x @ W_ref[...] + b_ref[...], 0.0)
        tf_ref[...] = t
        e = jnp.exp(t @ wp_ref[...])
        z_ref[:, 0:D] = t * e
        z_ref[:, D:AUG] = jnp.broadcast_to(e, (t.shape[0], AUG - D))

    one(hs_ref, Whs_ref, bhs_ref, wphs_ref, tfhs_ref, zhs_ref)
    one(hf_ref, Whf_ref, bhf_ref, wphf_ref, tfhf_ref, zhf_ref)


def _tc_stage(hs, hf, W_hs, b_hs, W_hf, b_hf, wp_hs, wp_hf):
    row_spec = pl.BlockSpec((_BLK, D), lambda i: (i, 0))
    full = lambda shape: pl.BlockSpec(shape, lambda i: (0, 0))
    return pl.pallas_call(
        _tc_body,
        grid=(_GRID,),
        in_specs=[row_spec, row_spec,
                  full((D, D)), full((1, D)), full((D, D)), full((1, D)),
                  full((D, 1)), full((D, 1))],
        out_specs=[row_spec, row_spec,
                   pl.BlockSpec((_BLK, AUG), lambda i: (i, 0)),
                   pl.BlockSpec((_BLK, AUG), lambda i: (i, 0))],
        out_shape=[jax.ShapeDtypeStruct((N_NODES, D), jnp.float32),
                   jax.ShapeDtypeStruct((N_NODES, D), jnp.float32),
                   jax.ShapeDtypeStruct((N_NODES, AUG), jnp.float32),
                   jax.ShapeDtypeStruct((N_NODES, AUG), jnp.float32)],
    )(hs, hf, W_hs, b_hs, W_hf, b_hf, wp_hs, wp_hf)


_NS = 16
_CHUNK = 128
_EPT = L // _NS
_NCHUNK = _EPT // _CHUNK
_SPT = N_SEG // _NS
_DIVQ = 16
_NDIVQ = _SPT // _DIVQ


_NBUF = 2


def _sc_body(zhs_hbm, zhf_hbm, idx_hbm, seg_hbm, hophs_hbm, hophf_hbm,
             idx2d_v, seg2d_v, row_bufs, outq_v, acc_sh, g_sems):
    cid = lax.axis_index("c")

    @pl.when(cid == 0)
    def _():
        _sc_process(zhs_hbm, idx_hbm, seg_hbm, hophs_hbm,
                    idx2d_v, seg2d_v, row_bufs, outq_v, acc_sh, g_sems)

    @pl.when(cid == 1)
    def _():
        _sc_process(zhf_hbm, idx_hbm, seg_hbm, hophf_hbm,
                    idx2d_v, seg2d_v, row_bufs, outq_v, acc_sh, g_sems)


def _sc_process(tbl_hbm, idx_hbm, seg_hbm, hop_hbm,
                idx2d_v, seg2d_v, row_bufs, outq_v, acc_sh, g_sems):
    sid = lax.axis_index("s")
    z16 = jnp.zeros((16,), jnp.float32)

    pltpu.sync_copy(idx_hbm.at[pl.ds(sid * _NCHUNK, _NCHUNK)], idx2d_v)
    pltpu.sync_copy(seg_hbm.at[pl.ds(sid * _NCHUNK, _NCHUNK)], seg2d_v)

    rows0 = row_bufs[0]
    def zrow(i, _):
        r = i // (AUG // 16)
        k = i % (AUG // 16)
        rows0[r, pl.ds(k * 16, 16)] = z16
        return 0
    lax.fori_loop(0, _CHUNK * (AUG // 16), zrow, 0)
    def zstripe(q, _):
        pltpu.sync_copy(rows0, acc_sh.at[pl.ds((sid * (_SPT // _CHUNK) + q) * _CHUNK, _CHUNK)])
        return 0
    lax.fori_loop(0, _SPT // _CHUNK, zstripe, 0)
    plsc.subcore_barrier()

    def gather(j, b):
        return pltpu.async_copy(tbl_hbm.at[idx2d_v.at[j]], row_bufs[b],
                                g_sems[b])

    for b in range(_NBUF):
        gather(b, b)

    def step(g, _):
        for b in range(_NBUF):
            j = g * _NBUF + b
            pltpu.make_async_copy(tbl_hbm.at[idx2d_v.at[j]], row_bufs[b],
                                  g_sems[b]).wait()
            pltpu.sync_copy(row_bufs[b], acc_sh.at[seg2d_v.at[j]], add=True)
            @pl.when(j + _NBUF < _NCHUNK)
            def _():
                gather(j + _NBUF, b)
        return 0
    lax.fori_loop(0, _NCHUNK // _NBUF, step, 0)
    plsc.subcore_barrier()

    def divq(q, _):
        seg0 = sid * _SPT + q * _DIVQ
        pltpu.sync_copy(acc_sh.at[pl.ds(seg0, _DIVQ)], rows0.at[pl.ds(0, _DIVQ)])
        def seg_body(r, _):
            den_v = rows0[r, pl.ds(D, 16)] + 1e-30
            def col(k, _):
                outq_v[r, pl.ds(k * 16, 16)] = rows0[r, pl.ds(k * 16, 16)] / den_v
                return 0
            lax.fori_loop(0, D // 16, col, 0)
            return 0
        lax.fori_loop(0, _DIVQ, seg_body, 0)
        pltpu.sync_copy(outq_v, hop_hbm.at[pl.ds(seg0, _DIVQ)])
        return 0
    lax.fori_loop(0, _NDIVQ, divq, 0)


@functools.cache
def _sc_stage():
    return pl.kernel(
        _sc_body,
        out_type=[jax.ShapeDtypeStruct((N_SEG, D), jnp.float32),
                  jax.ShapeDtypeStruct((N_SEG, D), jnp.float32)],
        mesh=plsc.VectorSubcoreMesh(core_axis_name="c", subcore_axis_name="s"),
        scratch_types=[
            pltpu.VMEM((_NCHUNK, _CHUNK), jnp.int32),
            pltpu.VMEM((_NCHUNK, _CHUNK), jnp.int32),
            [pltpu.VMEM((_CHUNK, AUG), jnp.float32) for _ in range(_NBUF)],
            pltpu.VMEM((_DIVQ, D), jnp.float32),
            pltpu.VMEM_SHARED((N_SEG, AUG), jnp.float32),
            [pltpu.SemaphoreType.DMA for _ in range(_NBUF)],
        ],
        compiler_params=pltpu.CompilerParams(use_tc_tiling_on_sc=False),
    )


def kernel(hs, hf, flat_idx, segment_ids, W_hs, b_hs, W_hf, b_hf,
           w_pool_hs, w_pool_hf):
    idx = flat_idx.astype(jnp.int32)
    seg = segment_ids.astype(jnp.int32)
    tf_hs, tf_hf, z_hs, z_hf = _tc_stage(
        hs, hf, W_hs, b_hs.reshape(1, D), W_hf, b_hf.reshape(1, D),
        w_pool_hs.reshape(D, 1), w_pool_hf.reshape(D, 1))
    idx2d = idx.reshape(L // _CHUNK, _CHUNK)
    seg2d = seg.reshape(L // _CHUNK, _CHUNK)
    hop_hs, hop_hf = _sc_stage()(z_hs, z_hf, idx2d, seg2d)
    return tf_hs, tf_hf, hop_hs, hop_hf


# speedup vs baseline: 1.3921x; 1.0195x over previous
import functools

import jax
import jax.numpy as jnp
from jax import lax
from jax.experimental import pallas as pl
from jax.experimental.pallas import tpu as pltpu
from jax.experimental.pallas import tpu_sc as plsc

N_NODES = 50000
D = 128
N_SEG = 8192
L = 131072
AUG = 144

_BLK = 512
_GRID = (N_NODES + _BLK - 1) // _BLK


def _tc_body(hs_ref, hf_ref, Whs_ref, bhs_ref, Whf_ref, bhf_ref,
             wphs_ref, wphf_ref, tfhs_ref, tfhf_ref, zhs_ref, zhf_ref):
    def one(x_ref, W_ref, b_ref, wp_ref, tf_ref, z_ref):
        x = x_ref[...]
        t = x + jnp.maximum(x @ W_ref[...] + b_ref[...], 0.0)
        tf_ref[...] = t
        e = jnp.exp(t @ wp_ref[...])
        z_ref[:, 0:D] = t * e
        z_ref[:, D:AUG] = jnp.broadcast_to(e, (t.shape[0], AUG - D))

    one(hs_ref, Whs_ref, bhs_ref, wphs_ref, tfhs_ref, zhs_ref)
    one(hf_ref, Whf_ref, bhf_ref, wphf_ref, tfhf_ref, zhf_ref)


def _tc_stage(hs, hf, W_hs, b_hs, W_hf, b_hf, wp_hs, wp_hf):
    row_spec = pl.BlockSpec((_BLK, D), lambda i: (i, 0))
    full = lambda shape: pl.BlockSpec(shape, lambda i: (0, 0))
    return pl.pallas_call(
        _tc_body,
        grid=(_GRID,),
        in_specs=[row_spec, row_spec,
                  full((D, D)), full((1, D)), full((D, D)), full((1, D)),
                  full((D, 1)), full((D, 1))],
        out_specs=[row_spec, row_spec,
                   pl.BlockSpec((_BLK, AUG), lambda i: (i, 0)),
                   pl.BlockSpec((_BLK, AUG), lambda i: (i, 0))],
        out_shape=[jax.ShapeDtypeStruct((N_NODES, D), jnp.float32),
                   jax.ShapeDtypeStruct((N_NODES, D), jnp.float32),
                   jax.ShapeDtypeStruct((N_NODES, AUG), jnp.float32),
                   jax.ShapeDtypeStruct((N_NODES, AUG), jnp.float32)],
    )(hs, hf, W_hs, b_hs, W_hf, b_hf, wp_hs, wp_hf)


_NS = 16
_CHUNK = 64
_EPT = L // _NS
_NCHUNK = _EPT // _CHUNK
_SPT = N_SEG // _NS
_DIVQ = 16
_NDIVQ = _SPT // _DIVQ


_NBUF = 4


def _sc_body(zhs_hbm, zhf_hbm, idx_hbm, seg_hbm, hophs_hbm, hophf_hbm,
             idx2d_v, seg2d_v, row_bufs, outq_v, acc_sh, g_sems, s_sems):
    cid = lax.axis_index("c")

    @pl.when(cid == 0)
    def _():
        _sc_process(zhs_hbm, idx_hbm, seg_hbm, hophs_hbm,
                    idx2d_v, seg2d_v, row_bufs, outq_v, acc_sh, g_sems,
                    s_sems)

    @pl.when(cid == 1)
    def _():
        _sc_process(zhf_hbm, idx_hbm, seg_hbm, hophf_hbm,
                    idx2d_v, seg2d_v, row_bufs, outq_v, acc_sh, g_sems,
                    s_sems)


def _sc_process(tbl_hbm, idx_hbm, seg_hbm, hop_hbm,
                idx2d_v, seg2d_v, row_bufs, outq_v, acc_sh, g_sems, s_sems):
    sid = lax.axis_index("s")
    z16 = jnp.zeros((16,), jnp.float32)

    pltpu.sync_copy(idx_hbm.at[pl.ds(sid * _NCHUNK, _NCHUNK)], idx2d_v)
    pltpu.sync_copy(seg_hbm.at[pl.ds(sid * _NCHUNK, _NCHUNK)], seg2d_v)

    rows0 = row_bufs[0]
    def zrow(i, _):
        r = i // (AUG // 16)
        k = i % (AUG // 16)
        rows0[r, pl.ds(k * 16, 16)] = z16
        return 0
    lax.fori_loop(0, _CHUNK * (AUG // 16), zrow, 0)
    def zstripe(q, _):
        pltpu.sync_copy(rows0, acc_sh.at[pl.ds((sid * (_SPT // _CHUNK) + q) * _CHUNK, _CHUNK)])
        return 0
    lax.fori_loop(0, _SPT // _CHUNK, zstripe, 0)
    plsc.subcore_barrier()

    def gather(j, b):
        return pltpu.async_copy(tbl_hbm.at[idx2d_v.at[j]], row_bufs[b],
                                g_sems[b])

    def wait_gather(j, b):
        pltpu.make_async_copy(tbl_hbm.at[idx2d_v.at[j]], row_bufs[b],
                              g_sems[b]).wait()

    def scatter(j, b):
        return pltpu.async_copy(row_bufs[b], acc_sh.at[seg2d_v.at[j]],
                                s_sems[b], add=True)

    def wait_scatter(j, b):
        pltpu.make_async_copy(row_bufs[b], acc_sh.at[seg2d_v.at[j]],
                              s_sems[b]).wait()

    for b in range(_NBUF - 1):
        gather(b, b)

    def step(g4, _):
        for b in range(_NBUF):
            j = g4 * _NBUF + b
            wait_gather(j, b)
            if b == 0:
                @pl.when(g4 > 0)
                def _():
                    wait_scatter(j - 1, (b - 1) % _NBUF)
            else:
                wait_scatter(j - 1, b - 1)
            scatter(j, b)
            @pl.when(j + _NBUF - 1 < _NCHUNK)
            def _():
                gather(j + _NBUF - 1, (b + _NBUF - 1) % _NBUF)
        return 0
    lax.fori_loop(0, _NCHUNK // _NBUF, step, 0)
    wait_scatter(_NCHUNK - 1, _NBUF - 1)
    plsc.subcore_barrier()

    def divq(q, _):
        seg0 = sid * _SPT + q * _DIVQ
        pltpu.sync_copy(acc_sh.at[pl.ds(seg0, _DIVQ)], rows0.at[pl.ds(0, _DIVQ)])
        def seg_body(r, _):
            den_v = rows0[r, pl.ds(D, 16)] + 1e-30
            def col(k, _):
                outq_v[r, pl.ds(k * 16, 16)] = rows0[r, pl.ds(k * 16, 16)] / den_v
                return 0
            lax.fori_loop(0, D // 16, col, 0)
            return 0
        lax.fori_loop(0, _DIVQ, seg_body, 0)
        pltpu.sync_copy(outq_v, hop_hbm.at[pl.ds(seg0, _DIVQ)])
        return 0
    lax.fori_loop(0, _NDIVQ, divq, 0)


@functools.cache
def _sc_stage():
    return pl.kernel(
        _sc_body,
        out_type=[jax.ShapeDtypeStruct((N_SEG, D), jnp.float32),
                  jax.ShapeDtypeStruct((N_SEG, D), jnp.float32)],
        mesh=plsc.VectorSubcoreMesh(core_axis_name="c", subcore_axis_name="s"),
        scratch_types=[
            pltpu.VMEM((_NCHUNK, _CHUNK), jnp.int32),
            pltpu.VMEM((_NCHUNK, _CHUNK), jnp.int32),
            [pltpu.VMEM((_CHUNK, AUG), jnp.float32) for _ in range(_NBUF)],
            pltpu.VMEM((_DIVQ, D), jnp.float32),
            pltpu.VMEM_SHARED((N_SEG, AUG), jnp.float32),
            [pltpu.SemaphoreType.DMA for _ in range(_NBUF)],
            [pltpu.SemaphoreType.DMA for _ in range(_NBUF)],
        ],
        compiler_params=pltpu.CompilerParams(use_tc_tiling_on_sc=False),
    )


def kernel(hs, hf, flat_idx, segment_ids, W_hs, b_hs, W_hf, b_hf,
           w_pool_hs, w_pool_hf):
    idx = flat_idx.astype(jnp.int32)
    seg = segment_ids.astype(jnp.int32)
    tf_hs, tf_hf, z_hs, z_hf = _tc_stage(
        hs, hf, W_hs, b_hs.reshape(1, D), W_hf, b_hf.reshape(1, D),
        w_pool_hs.reshape(D, 1), w_pool_hf.reshape(D, 1))
    idx2d = idx.reshape(L // _CHUNK, _CHUNK)
    seg2d = seg.reshape(L // _CHUNK, _CHUNK)
    hop_hs, hop_hf = _sc_stage()(z_hs, z_hf, idx2d, seg2d)
    return tf_hs, tf_hf, hop_hs, hop_hf


# speedup vs baseline: 1.8590x; 1.3353x over previous
import functools

import jax
import jax.numpy as jnp
from jax import lax
from jax.experimental import pallas as pl
from jax.experimental.pallas import tpu as pltpu
from jax.experimental.pallas import tpu_sc as plsc

N_NODES = 50000
D = 128
N_SEG = 8192
L = 131072

_BLK = 512
_GRID = (N_NODES + _BLK - 1) // _BLK


def _tc_body(hs_ref, hf_ref, Whs_ref, bhs_ref, Whf_ref, bhf_ref,
             wphs_ref, wphf_ref, tfhs_ref, tfhf_ref, zhs_ref, zhf_ref,
             ehs_ref, ehf_ref):
    def one(x_ref, W_ref, b_ref, wp_ref, tf_ref, z_ref, e_ref):
        x = x_ref[...]
        t = x + jnp.maximum(x @ W_ref[...] + b_ref[...], 0.0)
        tf_ref[...] = t
        e = jnp.exp(t @ wp_ref[...])
        z_ref[...] = t * e
        e_ref[...] = e

    one(hs_ref, Whs_ref, bhs_ref, wphs_ref, tfhs_ref, zhs_ref, ehs_ref)
    one(hf_ref, Whf_ref, bhf_ref, wphf_ref, tfhf_ref, zhf_ref, ehf_ref)


def _tc_stage(hs, hf, W_hs, b_hs, W_hf, b_hf, wp_hs, wp_hf):
    row_spec = pl.BlockSpec((_BLK, D), lambda i: (i, 0))
    col_spec = pl.BlockSpec((_BLK, 1), lambda i: (i, 0))
    full = lambda shape: pl.BlockSpec(shape, lambda i: (0, 0))
    return pl.pallas_call(
        _tc_body,
        grid=(_GRID,),
        in_specs=[row_spec, row_spec,
                  full((D, D)), full((1, D)), full((D, D)), full((1, D)),
                  full((D, 1)), full((D, 1))],
        out_specs=[row_spec, row_spec, row_spec, row_spec,
                   col_spec, col_spec],
        out_shape=[jax.ShapeDtypeStruct((N_NODES, D), jnp.float32),
                   jax.ShapeDtypeStruct((N_NODES, D), jnp.float32),
                   jax.ShapeDtypeStruct((N_NODES, D), jnp.float32),
                   jax.ShapeDtypeStruct((N_NODES, D), jnp.float32),
                   jax.ShapeDtypeStruct((N_NODES, 1), jnp.float32),
                   jax.ShapeDtypeStruct((N_NODES, 1), jnp.float32)],
    )(hs, hf, W_hs, b_hs, W_hf, b_hf, wp_hs, wp_hf)


_NS = 16
_CHUNK = 64
_EPT = L // _NS
_NCHUNK = _EPT // _CHUNK
_SPT = N_SEG // _NS
_DIVQ = 16
_NDIVQ = _SPT // _DIVQ
_NBUF = 4
_DEN = 16 * 528


def _sc_body(zhs_hbm, zhf_hbm, ehs_hbm, ehf_hbm, idx_hbm, seg_hbm,
             hophs_hbm, hophf_hbm,
             idx2d_v, seg2d_v, row_bufs, e_bufs, z1d_v, den_v, outq_v,
             acc_sh, den_sh, e_sh, g_sems, eg_sems, s_sems, es_sems):
    cid = lax.axis_index("c")
    args = (idx2d_v, seg2d_v, row_bufs, e_bufs, z1d_v, den_v, outq_v,
            acc_sh, den_sh, e_sh, g_sems, eg_sems, s_sems, es_sems)

    @pl.when(cid == 0)
    def _():
        _sc_process(zhs_hbm, ehs_hbm, idx_hbm, seg_hbm, hophs_hbm, *args)

    @pl.when(cid == 1)
    def _():
        _sc_process(zhf_hbm, ehf_hbm, idx_hbm, seg_hbm, hophf_hbm, *args)


def _sc_process(tbl_hbm, e_hbm, idx_hbm, seg_hbm, hop_hbm,
                idx2d_v, seg2d_v, row_bufs, e_bufs, z1d_v, den_v, outq_v,
                acc_sh, den_sh, e_sh, g_sems, eg_sems, s_sems, es_sems):
    sid = lax.axis_index("s")
    z16 = jnp.zeros((16,), jnp.float32)

    pltpu.sync_copy(idx_hbm.at[pl.ds(sid * _NCHUNK, _NCHUNK)], idx2d_v)
    pltpu.sync_copy(seg_hbm.at[pl.ds(sid * _NCHUNK, _NCHUNK)], seg2d_v)

    @pl.when(sid == 0)
    def _():
        pltpu.sync_copy(e_hbm, e_sh)

    rows0 = row_bufs[0]
    def zrow(i, _):
        r = i // (D // 16)
        k = i % (D // 16)
        rows0[r, pl.ds(k * 16, 16)] = z16
        return 0
    lax.fori_loop(0, _CHUNK * (D // 16), zrow, 0)
    def zd(i, _):
        z1d_v[pl.ds(i * 16, 16)] = z16
        return 0
    lax.fori_loop(0, (_DEN // _NS) // 16, zd, 0)
    def zstripe(q, _):
        pltpu.sync_copy(rows0, acc_sh.at[pl.ds((sid * (_SPT // _CHUNK) + q) * _CHUNK, _CHUNK)])
        return 0
    lax.fori_loop(0, _SPT // _CHUNK, zstripe, 0)
    pltpu.sync_copy(z1d_v, den_sh.at[pl.ds(sid * (_DEN // _NS), _DEN // _NS)])
    plsc.subcore_barrier()

    def gathers(j, b):
        pltpu.async_copy(tbl_hbm.at[idx2d_v.at[j]], row_bufs[b], g_sems[b])
        pltpu.async_copy(e_sh.at[idx2d_v.at[j]], e_bufs[b], eg_sems[b])

    def wait_gathers(j, b):
        pltpu.make_async_copy(tbl_hbm.at[idx2d_v.at[j]], row_bufs[b],
                              g_sems[b]).wait()
        pltpu.make_async_copy(e_sh.at[idx2d_v.at[j]], e_bufs[b],
                              eg_sems[b]).wait()

    def scatters(j, b):
        pltpu.async_copy(row_bufs[b], acc_sh.at[seg2d_v.at[j]], s_sems[b],
                         add=True)
        pltpu.async_copy(e_bufs[b], den_sh.at[seg2d_v.at[j]], es_sems[b],
                         add=True)

    def wait_scatters(j, b):
        pltpu.make_async_copy(row_bufs[b], acc_sh.at[seg2d_v.at[j]],
                              s_sems[b]).wait()
        pltpu.make_async_copy(e_bufs[b], den_sh.at[seg2d_v.at[j]],
                              es_sems[b]).wait()

    for b in range(_NBUF - 1):
        gathers(b, b)

    def step(g4, _):
        for b in range(_NBUF):
            j = g4 * _NBUF + b
            wait_gathers(j, b)
            if b == 0:
                @pl.when(g4 > 0)
                def _():
                    wait_scatters(j - 1, (b - 1) % _NBUF)
            else:
                wait_scatters(j - 1, b - 1)
            scatters(j, b)
            @pl.when(j + _NBUF - 1 < _NCHUNK)
            def _():
                gathers(j + _NBUF - 1, (b + _NBUF - 1) % _NBUF)
        return 0
    lax.fori_loop(0, _NCHUNK // _NBUF, step, 0)
    wait_scatters(_NCHUNK - 1, _NBUF - 1)
    plsc.subcore_barrier()

    pltpu.sync_copy(den_sh.at[pl.ds(sid * _SPT, _SPT)], den_v.at[pl.ds(0, _SPT)])
    def divq(q, _):
        seg0 = sid * _SPT + q * _DIVQ
        pltpu.sync_copy(acc_sh.at[pl.ds(seg0, _DIVQ)], rows0.at[pl.ds(0, _DIVQ)])
        def seg_body(r, _):
            d0 = den_v[pl.ds(q * _DIVQ + r, 16)][0]
            den16 = z16 + (d0 + 1e-30)
            def col(k, _):
                outq_v[r, pl.ds(k * 16, 16)] = rows0[r, pl.ds(k * 16, 16)] / den16
                return 0
            lax.fori_loop(0, D // 16, col, 0)
            return 0
        lax.fori_loop(0, _DIVQ, seg_body, 0)
        pltpu.sync_copy(outq_v, hop_hbm.at[pl.ds(seg0, _DIVQ)])
        return 0
    lax.fori_loop(0, _NDIVQ, divq, 0)


@functools.cache
def _sc_stage():
    return pl.kernel(
        _sc_body,
        out_type=[jax.ShapeDtypeStruct((N_SEG, D), jnp.float32),
                  jax.ShapeDtypeStruct((N_SEG, D), jnp.float32)],
        mesh=plsc.VectorSubcoreMesh(core_axis_name="c", subcore_axis_name="s"),
        scratch_types=[
            pltpu.VMEM((_NCHUNK, _CHUNK), jnp.int32),
            pltpu.VMEM((_NCHUNK, _CHUNK), jnp.int32),
            [pltpu.VMEM((_CHUNK, D), jnp.float32) for _ in range(_NBUF)],
            [pltpu.VMEM((_CHUNK,), jnp.float32) for _ in range(_NBUF)],
            pltpu.VMEM((_DEN // _NS,), jnp.float32),
            pltpu.VMEM((_DEN // _NS,), jnp.float32),
            pltpu.VMEM((_DIVQ, D), jnp.float32),
            pltpu.VMEM_SHARED((N_SEG, D), jnp.float32),
            pltpu.VMEM_SHARED((_DEN,), jnp.float32),
            pltpu.VMEM_SHARED((N_NODES,), jnp.float32),
            [pltpu.SemaphoreType.DMA for _ in range(_NBUF)],
            [pltpu.SemaphoreType.DMA for _ in range(_NBUF)],
            [pltpu.SemaphoreType.DMA for _ in range(_NBUF)],
            [pltpu.SemaphoreType.DMA for _ in range(_NBUF)],
        ],
        compiler_params=pltpu.CompilerParams(use_tc_tiling_on_sc=False),
    )


def kernel(hs, hf, flat_idx, segment_ids, W_hs, b_hs, W_hf, b_hf,
           w_pool_hs, w_pool_hf):
    idx = flat_idx.astype(jnp.int32)
    seg = segment_ids.astype(jnp.int32)
    tf_hs, tf_hf, z_hs, z_hf, e_hs, e_hf = _tc_stage(
        hs, hf, W_hs, b_hs.reshape(1, D), W_hf, b_hf.reshape(1, D),
        w_pool_hs.reshape(D, 1), w_pool_hf.reshape(D, 1))
    idx2d = idx.reshape(L // _CHUNK, _CHUNK)
    seg2d = seg.reshape(L // _CHUNK, _CHUNK)
    hop_hs, hop_hf = _sc_stage()(z_hs, z_hf, e_hs.reshape(N_NODES),
                                 e_hf.reshape(N_NODES), idx2d, seg2d)
    return tf_hs, tf_hf, hop_hs, hop_hf
